# baseline (speedup 1.0000x reference)
import jax
import jax.numpy as jnp
from jax.experimental import pallas as pl
from jax.experimental.pallas import tpu as pltpu

TOPK = 8


def _router_body(x_ref, gw_ref, gb_ref, eb_ref, probs_out, idx_out, xm_scr):
    b = pl.program_id(0)
    nb = pl.num_programs(0)
    spatial = x_ref.shape[2]
    xm_scr[b, :] = jnp.sum(x_ref[0], axis=1) * (1.0 / spatial)

    @pl.when(b == nb - 1)
    def _finish():
        xm = xm_scr[...]
        nrows, nexp = xm.shape[0], gw_ref.shape[0]
        logits = jax.lax.dot_general(
            xm, gw_ref[...], (((1,), (1,)), ((), ())),
            preferred_element_type=jnp.float32)
        logits = logits + gb_ref[...]
        logits = jnp.clip(logits, -10.0, 10.0)
        lb = logits + eb_ref[...]
        m = jnp.max(lb, axis=1, keepdims=True)
        e = jnp.exp(lb - m)
        p = e / jnp.sum(e, axis=1, keepdims=True)
        p = jnp.clip(p, 1e-06, 1.0)
        iota = jax.lax.broadcasted_iota(jnp.int32, (nrows, nexp), 1)
        vals, idxs = [], []
        for _ in range(TOPK):
            mk = jnp.max(p, axis=1, keepdims=True)
            ik = jnp.min(jnp.where(p == mk, iota, nexp), axis=1, keepdims=True)
            vals.append(mk)
            idxs.append(ik)
            p = jnp.where(iota == ik, -jnp.inf, p)
        tv = jnp.concatenate(vals, axis=1)
        ti = jnp.concatenate(idxs, axis=1)
        tv = tv / (jnp.sum(tv, axis=1, keepdims=True) + 1e-08)
        probs_out[...] = tv
        idx_out[...] = ti


def kernel(x, gate_w, gate_b, expert_bias):
    B, C, H, W = x.shape
    E = gate_w.shape[0]
    xr = x.reshape(B, C, H * W)
    gb = gate_b.reshape(1, E)
    eb = expert_bias.reshape(1, E)

    probs, idx = pl.pallas_call(
        _router_body,
        grid=(B,),
        in_specs=[
            pl.BlockSpec((1, C, H * W), lambda b: (b, 0, 0)),
            pl.BlockSpec((E, C), lambda b: (0, 0)),
            pl.BlockSpec((1, E), lambda b: (0, 0)),
            pl.BlockSpec((1, E), lambda b: (0, 0)),
        ],
        out_specs=[
            pl.BlockSpec((B, TOPK), lambda b: (0, 0)),
            pl.BlockSpec((B, TOPK), lambda b: (0, 0)),
        ],
        out_shape=[
            jax.ShapeDtypeStruct((B, TOPK), jnp.float32),
            jax.ShapeDtypeStruct((B, TOPK), jnp.int32),
        ],
        scratch_shapes=[pltpu.VMEM((B, C), jnp.float32)],
    )(xr, gate_w, gb, eb)

    loss = jnp.zeros((), dtype=jnp.float32)
    return (probs, idx, loss)
